# baseline (speedup 1.0000x reference)
import functools

import jax
import jax.numpy as jnp
from jax import lax
from jax.experimental import pallas as pl
from jax.experimental.pallas import tpu as pltpu
from jax.experimental.pallas import tpu_sc as plsc

TOPK = 16
Q = 1024
D = 64
N = 100000
QT = 256
KC = 4096
NPAD = 102400
NKC = NPAD // KC
NQT = Q // QT
G = 128
NG = NPAD // G
GPC = KC // G
B = Q * TOPK
NW = 32
BPW = B // NW
IROWS = B // 128
RPW = IROWS // NW
QT3 = 256
NQT3 = Q // QT3
IMAX = 2**31 - 1
QSHIFT = 10


def _score_body(q2_ref, kt_ref, s_ref, fid_ref, m_ref):
    qt = pl.program_id(0)
    kc = pl.program_id(1)
    q2 = q2_ref[...]
    kt = kt_ref[...].T
    qsq = 0.25 * jnp.sum(q2 * q2, axis=1, keepdims=True)
    ksq = jnp.sum(kt * kt, axis=0, keepdims=True)
    dots2 = jnp.dot(q2, kt, preferred_element_type=jnp.float32)
    s = (dots2 - qsq) - ksq
    for g in range(GPC):
        s_ref[g] = s[:, g * G:(g + 1) * G]
    gm = jnp.concatenate(
        [jnp.max(s[:, g * G:(g + 1) * G], axis=1, keepdims=True)
         for g in range(GPC)], axis=1)
    m_ref[pl.ds(kc * GPC, GPC), :] = gm.T

    @pl.when(kc == NKC - 1)
    def _():
        mt = m_ref[...]
        rows = lax.broadcasted_iota(jnp.int32, (NG, QT), 0)
        sels = []
        for _ in range(TOPK):
            mx = jnp.max(mt, axis=0, keepdims=True)
            cand = jnp.where(mt == mx, rows, IMAX)
            sel = jnp.min(cand, axis=0, keepdims=True)
            sels.append(sel)
            mt = jnp.where(rows == sel, -jnp.inf, mt)
        gid = jnp.concatenate(sels, axis=0)
        qrow = qt * QT + lax.broadcasted_iota(jnp.int32, (TOPK, QT), 1)
        fid_ref[...] = gid * Q + qrow


def _final_body(c_ref, fid_ref, vals_ref, idx_ref):
    c = c_ref[...]
    fid = fid_ref[...]
    kbase = (fid >> QSHIFT) * G
    lane = lax.broadcasted_iota(jnp.int32, (QT3, G), 1)
    idx2 = jnp.concatenate(
        [kbase[:, s:s + 1] + lane for s in range(TOPK)], axis=1)
    ext_s, ext_i = c, idx2
    new_v, new_i = [], []
    for _ in range(TOPK):
        m = jnp.max(ext_s, axis=1, keepdims=True)
        cand = jnp.where(ext_s == m, ext_i, IMAX)
        sel = jnp.min(cand, axis=1, keepdims=True)
        new_v.append(m)
        new_i.append(sel)
        ext_s = jnp.where(ext_i == sel, -jnp.inf, ext_s)
    vals_ref[...] = jnp.concatenate(new_v, axis=1)
    idx_ref[...] = jnp.concatenate(new_i, axis=1)


_GATHER_SC = []


def _get_gather_sc():
    if not _GATHER_SC:
        mesh = plsc.VectorSubcoreMesh(core_axis_name="c", subcore_axis_name="s")

        @functools.partial(
            pl.kernel,
            out_type=jax.ShapeDtypeStruct((B, G), jnp.float32),
            mesh=mesh,
            scratch_types=[
                pltpu.VMEM((RPW, 128), jnp.int32),
                pltpu.VMEM((BPW, G), jnp.float32),
                pltpu.SemaphoreType.DMA,
            ],
        )
        def _gather_sc(table_hbm, idx_hbm, out_hbm, idx_v, rows_v, sem):
            wid = lax.axis_index("s") * 2 + lax.axis_index("c")
            pltpu.sync_copy(idx_hbm.at[pl.ds(wid * RPW, RPW)], idx_v)
            copies = []
            for j in range(RPW):
                copies.append(pltpu.async_copy(
                    table_hbm.at[idx_v.at[j]],
                    rows_v.at[pl.ds(j * 128, 128)], sem))
            for cp in copies:
                cp.wait()
            pltpu.sync_copy(rows_v, out_hbm.at[pl.ds(wid * BPW, BPW)])

        _GATHER_SC.append(_gather_sc)
    return _GATHER_SC[0]


def kernel(queries, keys):
    pad = jnp.zeros((NPAD - N, D), jnp.float32).at[:, 0].set(30000.0)
    kt = jnp.concatenate([keys, pad], axis=0)
    q2 = queries * 2.0

    scores, fid_t = pl.pallas_call(
        _score_body,
        grid=(NQT, NKC),
        in_specs=[
            pl.BlockSpec((QT, D), lambda qt, kc: (qt, 0)),
            pl.BlockSpec((KC, D), lambda qt, kc: (kc, 0)),
        ],
        out_specs=[
            pl.BlockSpec((GPC, QT, G), lambda qt, kc: (kc, qt, 0)),
            pl.BlockSpec((TOPK, QT), lambda qt, kc: (0, qt)),
        ],
        out_shape=[
            jax.ShapeDtypeStruct((NG, Q, G), jnp.float32),
            jax.ShapeDtypeStruct((TOPK, Q), jnp.int32),
        ],
        scratch_shapes=[pltpu.VMEM((NG, QT), jnp.float32)],
    )(q2, kt)

    fid = fid_t.T
    table = scores.reshape(NG * Q, G)
    cand = _get_gather_sc()(table, fid.reshape(IROWS, 128))
    cand = cand.reshape(Q, TOPK * G)

    vals, idx = pl.pallas_call(
        _final_body,
        grid=(NQT3,),
        in_specs=[
            pl.BlockSpec((QT3, TOPK * G), lambda qt: (qt, 0)),
            pl.BlockSpec((QT3, TOPK), lambda qt: (qt, 0)),
        ],
        out_specs=[
            pl.BlockSpec((QT3, TOPK), lambda qt: (qt, 0)),
            pl.BlockSpec((QT3, TOPK), lambda qt: (qt, 0)),
        ],
        out_shape=[
            jax.ShapeDtypeStruct((Q, TOPK), jnp.float32),
            jax.ShapeDtypeStruct((Q, TOPK), jnp.int32),
        ],
    )(cand, fid)
    return vals, idx


# speedup vs baseline: 1.3093x; 1.3093x over previous
import functools

import jax
import jax.numpy as jnp
from jax import lax
from jax.experimental import pallas as pl
from jax.experimental.pallas import tpu as pltpu
from jax.experimental.pallas import tpu_sc as plsc

TOPK = 16
Q = 1024
D = 64
N = 100000
QT = 256
KC = 4096
NPAD = 102400
NKC = NPAD // KC
NQT = Q // QT
G = 128
NG = NPAD // G
GPC = KC // G
B = Q * TOPK
NW = 32
BPW = B // NW
IROWS = B // 128
RPW = IROWS // NW
QT3 = 256
NQT3 = Q // QT3
IMAX = 2**31 - 1
QSHIFT = 10


def _score_body(q2_ref, kt_ref, s_ref, fid_ref, m_ref, ksq_ref):
    kc = pl.program_id(0)
    qt = pl.program_id(1)
    q2 = q2_ref[...]
    kt = kt_ref[...]
    qsq = 0.25 * jnp.sum(q2 * q2, axis=1, keepdims=True)

    @pl.when(qt == 0)
    def _():
        ksq_ref[0:1, :] = jnp.sum(kt * kt, axis=0, keepdims=True)

    ksq = ksq_ref[0:1, :]
    dots2 = jnp.dot(q2, kt, preferred_element_type=jnp.float32)
    s = (dots2 - qsq) - ksq
    for g in range(GPC):
        s_ref[g] = s[:, g * G:(g + 1) * G]
    gm = jnp.concatenate(
        [jnp.max(s[:, g * G:(g + 1) * G], axis=1, keepdims=True)
         for g in range(GPC)], axis=1)
    m_ref[qt, pl.ds(kc * GPC, GPC), :] = gm.T

    @pl.when(kc == NKC - 1)
    def _():
        mt = m_ref[qt]
        rows = lax.broadcasted_iota(jnp.int32, (NG, QT), 0)
        sels = []
        for _ in range(TOPK):
            mx = jnp.max(mt, axis=0, keepdims=True)
            cand = jnp.where(mt == mx, rows, IMAX)
            sel = jnp.min(cand, axis=0, keepdims=True)
            sels.append(sel)
            mt = jnp.where(rows == sel, -jnp.inf, mt)
        gid = jnp.concatenate(sels, axis=0)
        qrow = qt * QT + lax.broadcasted_iota(jnp.int32, (TOPK, QT), 1)
        fid_ref[...] = gid * Q + qrow


def _final_body(c_ref, fid_ref, vals_ref, idx_ref):
    c = c_ref[...]
    fid = fid_ref[...]
    kbase = (fid >> QSHIFT) * G
    lane = lax.broadcasted_iota(jnp.int32, (QT3, G), 1)
    idx2 = jnp.concatenate(
        [kbase[:, s:s + 1] + lane for s in range(TOPK)], axis=1)
    ext_s, ext_i = c, idx2
    new_v, new_i = [], []
    for _ in range(TOPK):
        m = jnp.max(ext_s, axis=1, keepdims=True)
        cand = jnp.where(ext_s == m, ext_i, IMAX)
        sel = jnp.min(cand, axis=1, keepdims=True)
        new_v.append(m)
        new_i.append(sel)
        ext_s = jnp.where(ext_i == sel, -jnp.inf, ext_s)
    vals_ref[...] = jnp.concatenate(new_v, axis=1)
    idx_ref[...] = jnp.concatenate(new_i, axis=1)


_GATHER_SC = []


def _get_gather_sc():
    if not _GATHER_SC:
        mesh = plsc.VectorSubcoreMesh(core_axis_name="c", subcore_axis_name="s")

        @functools.partial(
            pl.kernel,
            out_type=jax.ShapeDtypeStruct((B, G), jnp.float32),
            mesh=mesh,
            scratch_types=[
                pltpu.VMEM((RPW, 128), jnp.int32),
                pltpu.VMEM((BPW, G), jnp.float32),
                pltpu.SemaphoreType.DMA,
            ],
        )
        def _gather_sc(table_hbm, idx_hbm, out_hbm, idx_v, rows_v, sem):
            wid = lax.axis_index("s") * 2 + lax.axis_index("c")
            pltpu.sync_copy(idx_hbm.at[pl.ds(wid * RPW, RPW)], idx_v)
            copies = []
            for j in range(RPW):
                copies.append(pltpu.async_copy(
                    table_hbm.at[idx_v.at[j]],
                    rows_v.at[pl.ds(j * 128, 128)], sem))
            for cp in copies:
                cp.wait()
            pltpu.sync_copy(rows_v, out_hbm.at[pl.ds(wid * BPW, BPW)])

        _GATHER_SC.append(_gather_sc)
    return _GATHER_SC[0]


def kernel(queries, keys):
    pad = jnp.zeros((NPAD - N, D), jnp.float32).at[:, 0].set(30000.0)
    kt = jnp.concatenate([keys, pad], axis=0).T
    q2 = queries * 2.0

    scores, fid_t = pl.pallas_call(
        _score_body,
        grid=(NKC, NQT),
        in_specs=[
            pl.BlockSpec((QT, D), lambda kc, qt: (qt, 0)),
            pl.BlockSpec((D, KC), lambda kc, qt: (0, kc)),
        ],
        out_specs=[
            pl.BlockSpec((GPC, QT, G), lambda kc, qt: (kc, qt, 0)),
            pl.BlockSpec((TOPK, QT), lambda kc, qt: (0, qt)),
        ],
        out_shape=[
            jax.ShapeDtypeStruct((NG, Q, G), jnp.float32),
            jax.ShapeDtypeStruct((TOPK, Q), jnp.int32),
        ],
        scratch_shapes=[pltpu.VMEM((NQT, NG, QT), jnp.float32),
                        pltpu.VMEM((8, KC), jnp.float32)],
    )(q2, kt)

    fid = fid_t.T
    table = scores.reshape(NG * Q, G)
    cand = _get_gather_sc()(table, fid.reshape(IROWS, 128))
    cand = cand.reshape(Q, TOPK * G)

    vals, idx = pl.pallas_call(
        _final_body,
        grid=(NQT3,),
        in_specs=[
            pl.BlockSpec((QT3, TOPK * G), lambda qt: (qt, 0)),
            pl.BlockSpec((QT3, TOPK), lambda qt: (qt, 0)),
        ],
        out_specs=[
            pl.BlockSpec((QT3, TOPK), lambda qt: (qt, 0)),
            pl.BlockSpec((QT3, TOPK), lambda qt: (qt, 0)),
        ],
        out_shape=[
            jax.ShapeDtypeStruct((Q, TOPK), jnp.float32),
            jax.ShapeDtypeStruct((Q, TOPK), jnp.int32),
        ],
    )(cand, fid)
    return vals, idx


# speedup vs baseline: 1.4477x; 1.1057x over previous
import functools

import jax
import jax.numpy as jnp
from jax import lax
from jax.experimental import pallas as pl
from jax.experimental.pallas import tpu as pltpu
from jax.experimental.pallas import tpu_sc as plsc

TOPK = 16
Q = 1024
D = 64
N = 100000
QT = 512
KC = 4096
NPAD = 102400
NKC = NPAD // KC
NQT = Q // QT
G = 128
NG = NPAD // G
GPC = KC // G
B = Q * TOPK
NW = 32
BPW = B // NW
IROWS = B // 128
RPW = IROWS // NW
QT3 = 256
NQT3 = Q // QT3
IMAX = 2**31 - 1
QSHIFT = 10


def _score_body(q2_ref, kt_ref, s_ref, fid_ref, m_ref, ksq_ref):
    kc = pl.program_id(0)
    qt = pl.program_id(1)
    q2 = q2_ref[...]
    kt = kt_ref[...]
    qsq = 0.25 * jnp.sum(q2 * q2, axis=1, keepdims=True)

    @pl.when(qt == 0)
    def _():
        ksq_ref[0:1, :] = jnp.sum(kt * kt, axis=0, keepdims=True)

    ksq = ksq_ref[0:1, :]
    dots2 = jnp.dot(q2, kt, preferred_element_type=jnp.float32)
    s = (dots2 - qsq) - ksq
    for g in range(GPC):
        s_ref[g] = s[:, g * G:(g + 1) * G]
    gm = jnp.concatenate(
        [jnp.max(s[:, g * G:(g + 1) * G], axis=1, keepdims=True)
         for g in range(GPC)], axis=1)
    m_ref[qt, pl.ds(kc * GPC, GPC), :] = gm.T

    @pl.when(kc == NKC - 1)
    def _():
        mt = m_ref[qt]
        rows = lax.broadcasted_iota(jnp.int32, (NG, QT), 0)
        sels = []
        for _ in range(TOPK):
            mx = jnp.max(mt, axis=0, keepdims=True)
            cand = jnp.where(mt == mx, rows, IMAX)
            sel = jnp.min(cand, axis=0, keepdims=True)
            sels.append(sel)
            mt = jnp.where(rows == sel, -jnp.inf, mt)
        gid = jnp.concatenate(sels, axis=0)
        qrow = qt * QT + lax.broadcasted_iota(jnp.int32, (TOPK, QT), 1)
        fid_ref[...] = gid * Q + qrow


def _final_body(c_ref, fid_ref, vals_ref, idx_ref):
    c = c_ref[...]
    fid = fid_ref[...]
    kbase = (fid >> QSHIFT) * G
    lane = lax.broadcasted_iota(jnp.int32, (QT3, G), 1)
    idx2 = jnp.concatenate(
        [kbase[:, s:s + 1] + lane for s in range(TOPK)], axis=1)
    ext_s, ext_i = c, idx2
    new_v, new_i = [], []
    for _ in range(TOPK):
        m = jnp.max(ext_s, axis=1, keepdims=True)
        cand = jnp.where(ext_s == m, ext_i, IMAX)
        sel = jnp.min(cand, axis=1, keepdims=True)
        new_v.append(m)
        new_i.append(sel)
        ext_s = jnp.where(ext_i == sel, -jnp.inf, ext_s)
    vals_ref[...] = jnp.concatenate(new_v, axis=1)
    idx_ref[...] = jnp.concatenate(new_i, axis=1)


_GATHER_SC = []


def _get_gather_sc():
    if not _GATHER_SC:
        mesh = plsc.VectorSubcoreMesh(core_axis_name="c", subcore_axis_name="s")

        @functools.partial(
            pl.kernel,
            out_type=jax.ShapeDtypeStruct((B, G), jnp.float32),
            mesh=mesh,
            scratch_types=[
                pltpu.VMEM((RPW, 128), jnp.int32),
                pltpu.VMEM((BPW, G), jnp.float32),
                pltpu.SemaphoreType.DMA,
            ],
        )
        def _gather_sc(table_hbm, idx_hbm, out_hbm, idx_v, rows_v, sem):
            wid = lax.axis_index("s") * 2 + lax.axis_index("c")
            pltpu.sync_copy(idx_hbm.at[pl.ds(wid * RPW, RPW)], idx_v)
            copies = []
            for j in range(RPW):
                copies.append(pltpu.async_copy(
                    table_hbm.at[idx_v.at[j]],
                    rows_v.at[pl.ds(j * 128, 128)], sem))
            for cp in copies:
                cp.wait()
            pltpu.sync_copy(rows_v, out_hbm.at[pl.ds(wid * BPW, BPW)])

        _GATHER_SC.append(_gather_sc)
    return _GATHER_SC[0]


def kernel(queries, keys):
    pad = jnp.zeros((NPAD - N, D), jnp.float32).at[:, 0].set(30000.0)
    kt = jnp.concatenate([keys, pad], axis=0).T
    q2 = queries * 2.0

    scores, fid_t = pl.pallas_call(
        _score_body,
        grid=(NKC, NQT),
        in_specs=[
            pl.BlockSpec((QT, D), lambda kc, qt: (qt, 0)),
            pl.BlockSpec((D, KC), lambda kc, qt: (0, kc)),
        ],
        out_specs=[
            pl.BlockSpec((GPC, QT, G), lambda kc, qt: (kc, qt, 0)),
            pl.BlockSpec((TOPK, QT), lambda kc, qt: (0, qt)),
        ],
        out_shape=[
            jax.ShapeDtypeStruct((NG, Q, G), jnp.float32),
            jax.ShapeDtypeStruct((TOPK, Q), jnp.int32),
        ],
        scratch_shapes=[pltpu.VMEM((NQT, NG, QT), jnp.float32),
                        pltpu.VMEM((8, KC), jnp.float32)],
    )(q2, kt)

    fid = fid_t.T
    table = scores.reshape(NG * Q, G)
    cand = _get_gather_sc()(table, fid.reshape(IROWS, 128))
    cand = cand.reshape(Q, TOPK * G)

    vals, idx = pl.pallas_call(
        _final_body,
        grid=(NQT3,),
        in_specs=[
            pl.BlockSpec((QT3, TOPK * G), lambda qt: (qt, 0)),
            pl.BlockSpec((QT3, TOPK), lambda qt: (qt, 0)),
        ],
        out_specs=[
            pl.BlockSpec((QT3, TOPK), lambda qt: (qt, 0)),
            pl.BlockSpec((QT3, TOPK), lambda qt: (qt, 0)),
        ],
        out_shape=[
            jax.ShapeDtypeStruct((Q, TOPK), jnp.float32),
            jax.ShapeDtypeStruct((Q, TOPK), jnp.int32),
        ],
    )(cand, fid)
    return vals, idx


# speedup vs baseline: 1.5555x; 1.0744x over previous
import functools

import jax
import jax.numpy as jnp
from jax import lax
from jax.experimental import pallas as pl
from jax.experimental.pallas import tpu as pltpu
from jax.experimental.pallas import tpu_sc as plsc

TOPK = 16
Q = 1024
D = 64
N = 100000
QT = 1024
KC = 4096
NPAD = 102400
NKC = NPAD // KC
NQT = Q // QT
G = 128
NG = NPAD // G
GPC = KC // G
B = Q * TOPK
NW = 32
BPW = B // NW
IROWS = B // 128
RPW = IROWS // NW
QT3 = 256
NQT3 = Q // QT3
IMAX = 2**31 - 1
QSHIFT = 10


def _score_body(q2_ref, kt_ref, s_ref, fid_ref, m_ref, ksq_ref):
    kc = pl.program_id(0)
    qt = pl.program_id(1)
    q2 = q2_ref[...]
    kt = kt_ref[...]
    qsq = 0.25 * jnp.sum(q2 * q2, axis=1, keepdims=True)

    @pl.when(qt == 0)
    def _():
        ksq_ref[0:1, :] = jnp.sum(kt * kt, axis=0, keepdims=True)

    ksq = ksq_ref[0:1, :]
    dots2 = jnp.dot(q2, kt, preferred_element_type=jnp.float32)
    s = (dots2 - qsq) - ksq
    for g in range(GPC):
        s_ref[g] = s[:, g * G:(g + 1) * G]
    gm = jnp.concatenate(
        [jnp.max(s[:, g * G:(g + 1) * G], axis=1, keepdims=True)
         for g in range(GPC)], axis=1)
    m_ref[qt, pl.ds(kc * GPC, GPC), :] = gm.T

    @pl.when(kc == NKC - 1)
    def _():
        mt = m_ref[qt]
        rows = lax.broadcasted_iota(jnp.int32, (NG, QT), 0)
        sels = []
        for _ in range(TOPK):
            mx = jnp.max(mt, axis=0, keepdims=True)
            cand = jnp.where(mt == mx, rows, IMAX)
            sel = jnp.min(cand, axis=0, keepdims=True)
            sels.append(sel)
            mt = jnp.where(rows == sel, -jnp.inf, mt)
        gid = jnp.concatenate(sels, axis=0)
        qrow = qt * QT + lax.broadcasted_iota(jnp.int32, (TOPK, QT), 1)
        fid_ref[...] = gid * Q + qrow


def _final_body(c_ref, fid_ref, vals_ref, idx_ref):
    c = c_ref[...]
    fid = fid_ref[...]
    kbase = (fid >> QSHIFT) * G
    lane = lax.broadcasted_iota(jnp.int32, (QT3, G), 1)
    idx2 = jnp.concatenate(
        [kbase[:, s:s + 1] + lane for s in range(TOPK)], axis=1)
    ext_s, ext_i = c, idx2
    new_v, new_i = [], []
    for _ in range(TOPK):
        m = jnp.max(ext_s, axis=1, keepdims=True)
        cand = jnp.where(ext_s == m, ext_i, IMAX)
        sel = jnp.min(cand, axis=1, keepdims=True)
        new_v.append(m)
        new_i.append(sel)
        ext_s = jnp.where(ext_i == sel, -jnp.inf, ext_s)
    vals_ref[...] = jnp.concatenate(new_v, axis=1)
    idx_ref[...] = jnp.concatenate(new_i, axis=1)


_GATHER_SC = []


def _get_gather_sc():
    if not _GATHER_SC:
        mesh = plsc.VectorSubcoreMesh(core_axis_name="c", subcore_axis_name="s")

        @functools.partial(
            pl.kernel,
            out_type=jax.ShapeDtypeStruct((B, G), jnp.float32),
            mesh=mesh,
            scratch_types=[
                pltpu.VMEM((RPW, 128), jnp.int32),
                pltpu.VMEM((BPW, G), jnp.float32),
                pltpu.SemaphoreType.DMA,
            ],
        )
        def _gather_sc(table_hbm, idx_hbm, out_hbm, idx_v, rows_v, sem):
            wid = lax.axis_index("s") * 2 + lax.axis_index("c")
            pltpu.sync_copy(idx_hbm.at[pl.ds(wid * RPW, RPW)], idx_v)
            copies = []
            for j in range(RPW):
                copies.append(pltpu.async_copy(
                    table_hbm.at[idx_v.at[j]],
                    rows_v.at[pl.ds(j * 128, 128)], sem))
            for cp in copies:
                cp.wait()
            pltpu.sync_copy(rows_v, out_hbm.at[pl.ds(wid * BPW, BPW)])

        _GATHER_SC.append(_gather_sc)
    return _GATHER_SC[0]


def kernel(queries, keys):
    pad = jnp.zeros((NPAD - N, D), jnp.float32).at[:, 0].set(30000.0)
    kt = jnp.concatenate([keys, pad], axis=0).T
    q2 = queries * 2.0

    scores, fid_t = pl.pallas_call(
        _score_body,
        grid=(NKC, NQT),
        in_specs=[
            pl.BlockSpec((QT, D), lambda kc, qt: (qt, 0)),
            pl.BlockSpec((D, KC), lambda kc, qt: (0, kc)),
        ],
        out_specs=[
            pl.BlockSpec((GPC, QT, G), lambda kc, qt: (kc, qt, 0)),
            pl.BlockSpec((TOPK, QT), lambda kc, qt: (0, qt)),
        ],
        out_shape=[
            jax.ShapeDtypeStruct((NG, Q, G), jnp.float32),
            jax.ShapeDtypeStruct((TOPK, Q), jnp.int32),
        ],
        scratch_shapes=[pltpu.VMEM((NQT, NG, QT), jnp.float32),
                        pltpu.VMEM((8, KC), jnp.float32)],
    )(q2, kt)

    fid = fid_t.T
    table = scores.reshape(NG * Q, G)
    cand = _get_gather_sc()(table, fid.reshape(IROWS, 128))
    cand = cand.reshape(Q, TOPK * G)

    vals, idx = pl.pallas_call(
        _final_body,
        grid=(NQT3,),
        in_specs=[
            pl.BlockSpec((QT3, TOPK * G), lambda qt: (qt, 0)),
            pl.BlockSpec((QT3, TOPK), lambda qt: (qt, 0)),
        ],
        out_specs=[
            pl.BlockSpec((QT3, TOPK), lambda qt: (qt, 0)),
            pl.BlockSpec((QT3, TOPK), lambda qt: (qt, 0)),
        ],
        out_shape=[
            jax.ShapeDtypeStruct((Q, TOPK), jnp.float32),
            jax.ShapeDtypeStruct((Q, TOPK), jnp.int32),
        ],
    )(cand, fid)
    return vals, idx
